# Initial kernel scaffold; baseline (speedup 1.0000x reference)
import jax
import jax.numpy as jnp
from jax.experimental import pallas as pl


def kernel(k, v):
    raise NotImplementedError("write your pallas kernel here")


# speedup vs baseline: 4.1812x; 4.1812x over previous
import functools

import jax
import jax.numpy as jnp
from jax import lax
from jax.experimental import pallas as pl
from jax.experimental.pallas import tpu as pltpu
from jax.experimental.pallas import tpu_sc as plsc

S, H, W, D = 2048, 12, 32, 64
PAD = W

_info = plsc.get_sparse_core_info()
NC, NS, NL = _info.num_cores, _info.num_subcores, _info.num_lanes
NW = NC * NS
CT = 32
NCHUNK = S // CT


def _body(k_hbm, v_hbm, z_hbm, kw_hbm, vw_hbm, stage, sem_g, sem_s):
    wid = lax.axis_index("s") * NC + lax.axis_index("c")

    def run_chunk(src_hbm, dst_hbm, c, edge_chunk):
        t0 = c * CT

        if edge_chunk:
            @pl.when(c == 0)
            def _():
                pltpu.sync_copy(z_hbm, stage.at[:, pl.ds(0, PAD), :])
                for h in range(H):
                    pltpu.async_copy(src_hbm.at[h, pl.ds(0, CT), :],
                                     stage.at[h, pl.ds(PAD, CT), :], sem_g)
                for h in range(H):
                    pltpu.make_async_copy(src_hbm.at[h, pl.ds(0, CT), :],
                                          stage.at[h, pl.ds(PAD, CT), :],
                                          sem_g).wait()

            @pl.when(c > 0)
            def _():
                for h in range(H):
                    pltpu.async_copy(src_hbm.at[h, pl.ds(t0 - PAD, CT + PAD), :],
                                     stage.at[h], sem_g)
                for h in range(H):
                    pltpu.make_async_copy(
                        src_hbm.at[h, pl.ds(t0 - PAD, CT + PAD), :],
                        stage.at[h], sem_g).wait()

            @pl.when(c == 0)
            def _():
                def edge_step(t, carry):
                    r = PAD + t
                    for h in range(H):
                        for j in range(D // NL):
                            stage[h, t, pl.ds(j * NL, NL)] = (
                                stage[h, r, pl.ds(j * NL, NL)])
                    pltpu.sync_copy(stage.at[:, pl.ds(0, W), :], dst_hbm.at[t])
                    return carry

                lax.fori_loop(0, W - 1, edge_step, 0)
        else:
            for h in range(H):
                pltpu.async_copy(src_hbm.at[h, pl.ds(t0 - PAD, CT + PAD), :],
                                 stage.at[h], sem_g)
            for h in range(H):
                pltpu.make_async_copy(src_hbm.at[h, pl.ds(t0 - PAD, CT + PAD), :],
                                      stage.at[h], sem_g).wait()

        def scat(i, carry):
            t = t0 + i

            @pl.when(t >= W - 1)
            def _():
                pltpu.async_copy(stage.at[:, pl.ds(i + 1, W), :],
                                 dst_hbm.at[t], sem_s)

            return carry

        def drain(i, carry):
            t = t0 + i

            @pl.when(t >= W - 1)
            def _():
                pltpu.make_async_copy(stage.at[:, pl.ds(i + 1, W), :],
                                      dst_hbm.at[t], sem_s).wait()

            return carry

        lax.fori_loop(0, CT, scat, 0)
        lax.fori_loop(0, CT, drain, 0)

    for src_hbm, dst_hbm in ((k_hbm, kw_hbm), (v_hbm, vw_hbm)):
        run_chunk(src_hbm, dst_hbm, wid, True)
        run_chunk(src_hbm, dst_hbm, wid + NW, False)


@jax.jit
def _unfold(kt, vt, z):
    fn = functools.partial(
        pl.kernel,
        out_type=(
            jax.ShapeDtypeStruct((S, H, W, D), jnp.float32),
            jax.ShapeDtypeStruct((S, H, W, D), jnp.float32),
        ),
        mesh=plsc.VectorSubcoreMesh(core_axis_name="c", subcore_axis_name="s"),
        scratch_types=[
            pltpu.VMEM((H, CT + PAD, D), jnp.float32),
            pltpu.SemaphoreType.DMA,
            pltpu.SemaphoreType.DMA,
        ],
    )(_body)
    return fn(kt, vt, z)


def kernel(k, v):
    kt = jnp.transpose(k[0], (1, 0, 2))
    vt = jnp.transpose(v[0], (1, 0, 2))
    kw, vw = _unfold(kt, vt, jnp.zeros((H, PAD, D), jnp.float32))
    return kw[None], vw[None]
